# Initial kernel scaffold; baseline (speedup 1.0000x reference)
import functools

import jax
import jax.numpy as jnp
from jax import lax
from jax.experimental import pallas as pl
from jax.experimental.pallas import tpu as pltpu
from jax.experimental.pallas import tpu_sc as plsc

N_NAMES = 9
VOCAB = 124
D = 128
N = 100000
B = 128
NBLK = (N + B - 1) // B
NW = 32

_mesh = plsc.VectorSubcoreMesh(core_axis_name="c", subcore_axis_name="s")


@functools.partial(
    pl.kernel,
    out_type=jax.ShapeDtypeStruct((N, D), jnp.float32),
    mesh=_mesh,
    scratch_types=[
        pltpu.VMEM((N_NAMES, B), jnp.int32),
        pltpu.VMEM((B, D), jnp.float32),
        pltpu.SemaphoreType.DMA,
        pltpu.SemaphoreType.DMA,
    ],
)
def _embed_sum(nf_hbm, tables_hbm, out_hbm, idx_v, acc_v, sem_i, sem_g):
    cid = lax.axis_index("c")
    sid = lax.axis_index("s")
    wid = sid * 2 + cid

    def body(k, carry):
        b = wid + k * NW
        base = jnp.where(b == NBLK - 1, N - B, b * B)
        for i in range(N_NAMES):
            pltpu.async_copy(nf_hbm.at[i, pl.ds(base, B)], idx_v.at[i], sem_i)
        for i in range(N_NAMES):
            pltpu.make_async_copy(
                nf_hbm.at[i, pl.ds(base, B)], idx_v.at[i], sem_i
            ).wait()
        pltpu.async_copy(tables_hbm.at[0].at[idx_v.at[0]], acc_v, sem_g).wait()
        for i in range(1, N_NAMES):
            pltpu.async_copy(
                tables_hbm.at[i].at[idx_v.at[i]], acc_v, sem_g, add=True
            ).wait()
        pltpu.sync_copy(acc_v, out_hbm.at[pl.ds(base, B)])
        return carry

    nmine = (NBLK - wid + NW - 1) // NW
    lax.fori_loop(0, nmine, body, 0)


def kernel(node_features, tables):
    return _embed_sum(node_features, tables)


# speedup vs baseline: 4.3540x; 4.3540x over previous
import functools

import jax
import jax.numpy as jnp
from jax import lax
from jax.experimental import pallas as pl
from jax.experimental.pallas import tpu as pltpu
from jax.experimental.pallas import tpu_sc as plsc

N_NAMES = 9
VOCAB = 124
D = 128
N = 100000
B = 128
NBLK = (N + B - 1) // B
NW = 32

_mesh = plsc.VectorSubcoreMesh(core_axis_name="c", subcore_axis_name="s")


@functools.partial(
    pl.kernel,
    out_type=jax.ShapeDtypeStruct((N, D), jnp.float32),
    mesh=_mesh,
    scratch_types=[
        pltpu.VMEM((N_NAMES, B), jnp.int32),
        pltpu.VMEM((B, D), jnp.float32),
        pltpu.SemaphoreType.DMA,
        pltpu.SemaphoreType.DMA,
    ],
)
def _embed_sum(nf_hbm, tables_hbm, out_hbm, idx_v, acc_v, sem_i, sem_g):
    cid = lax.axis_index("c")
    sid = lax.axis_index("s")
    wid = sid * 2 + cid

    def body(k, carry):
        b = wid + k * NW
        base = jnp.where(b == NBLK - 1, N - B, b * B)
        for i in range(N_NAMES):
            pltpu.async_copy(nf_hbm.at[pl.ds(i * N + base, B)], idx_v.at[i], sem_i)
        for i in range(N_NAMES):
            pltpu.make_async_copy(
                nf_hbm.at[pl.ds(i * N + base, B)], idx_v.at[i], sem_i
            ).wait()
        pltpu.async_copy(tables_hbm.at[0].at[idx_v.at[0]], acc_v, sem_g).wait()
        for i in range(1, N_NAMES):
            pltpu.async_copy(
                tables_hbm.at[i].at[idx_v.at[i]], acc_v, sem_g, add=True
            ).wait()
        pltpu.sync_copy(acc_v, out_hbm.at[pl.ds(base, B)])
        return carry

    nmine = (NBLK - wid + NW - 1) // NW
    lax.fori_loop(0, nmine, body, 0)


def kernel(node_features, tables):
    return _embed_sum(node_features.reshape(-1), tables)
